# baseline (speedup 1.0000x reference)
import functools
import math

import numpy as np
import jax
import jax.numpy as jnp
from jax import lax
from jax.experimental import pallas as pl
from jax.experimental.pallas import tpu as pltpu
from jax.experimental.pallas import tpu_sc as plsc

H, W, C = 200, 176, 64
NROI = 5000
PH = PW = 7
GH = GW = 2
SAMP_PER_ROI = PH * PW * GH * GW
BINS = NROI * PH * PW
NTILES = 32
CHUNK_BINS = 32
CHUNK_SAMP = CHUNK_BINS * GH * GW
NGROUP = CHUNK_SAMP // 16
CHUNKS_TOTAL = -(-BINS // CHUNK_BINS)
CHUNKS_PER_TILE = -(-CHUNKS_TOTAL // NTILES)
BINS_PAD = CHUNKS_PER_TILE * NTILES * CHUNK_BINS
N_PAD = (BINS_PAD * GH * GW - 1) // SAMP_PER_ROI + 1
AB_PAD = 208

_f32 = jnp.float32
_i32 = jnp.int32


def _splat_i32(v):
    return jnp.full((16,), v, dtype=_i32)


def _coords_weights(samp_base, params_v, ab_v, idx_v, w_v):
    lane = lax.iota(_i32, 16)
    for g in range(NGROUP):
        s = samp_base + g * 16 + lane
        roi = lax.div(s, _i32(SAMP_PER_ROI))
        rem = s - roi * SAMP_PER_ROI

        def p(k):
            return plsc.load_gather(params_v, [_splat_i32(k), roi])

        ch, cw, co, si, bh, bw, rh2, rw2 = (p(k) for k in range(8))
        a = plsc.load_gather(ab_v, [_splat_i32(0), rem])
        b = plsc.load_gather(ab_v, [_splat_i32(1), rem])
        yy = a * bh - rh2
        xx = b * bw - rw2
        y = yy * co - xx * si + ch
        x = yy * si + xx * co + cw
        valid = (y > -1.0) & (y < float(H)) & (x > -1.0) & (x < float(W))
        yc = jnp.maximum(y, 0.0)
        xc = jnp.maximum(x, 0.0)
        yl = jnp.minimum(yc.astype(_i32), H - 1)
        xl = jnp.minimum(xc.astype(_i32), W - 1)
        ly = yc - yl.astype(_f32)
        lx = xc - xl.astype(_f32)
        hy = 1.0 - ly
        hx = 1.0 - lx
        vf = jnp.where(valid, _f32(0.25), _f32(0.0))
        idx_v[pl.ds(g * 16, 16)] = yl * W + xl
        s_local = g * 16 + lane
        b_local = lax.div(s_local, _i32(4))
        pos = b_local * 16 + (s_local - b_local * 4) * 4
        plsc.store_scatter(w_v, [pos], hy * hx * vf)
        plsc.store_scatter(w_v, [pos + 1], hy * lx * vf)
        plsc.store_scatter(w_v, [pos + 2], ly * hx * vf)
        plsc.store_scatter(w_v, [pos + 3], ly * lx * vf)


def _accumulate(rows_v, w_v, out_v):
    def bin_body(b, carry):
        accs = [jnp.zeros((16,), _f32) for _ in range(4)]
        wvec = w_v[pl.ds(b * 16, 16)]
        for q in range(GH * GW):
            s = b * 4 + q
            for k in range(4):
                w = wvec[q * 4 + k]
                for h in range(2):
                    v16 = rows_v[s, pl.ds(k * 32 + h * 16, 16)]
                    vbf = plsc.bitcast(v16, jnp.bfloat16)
                    va, vb = plsc.unpack(vbf, format=plsc.PackFormat.INTERLEAVED)
                    accs[2 * h] = accs[2 * h] + w * va
                    accs[2 * h + 1] = accs[2 * h + 1] + w * vb
        for c4 in range(4):
            out_v[b, pl.ds(c4 * 16, 16)] = accs[c4]
        return carry

    lax.fori_loop(0, CHUNK_BINS, bin_body, 0)


def _sc_body(patch_hbm, params_hbm, ab_hbm, out_hbm,
             params_v, ab_v, idx0, idx1, w0, w1, rows0, rows1,
             outb0, outb1, sem0, sem1, osem0, osem1):
    pltpu.sync_copy(params_hbm, params_v)
    pltpu.sync_copy(ab_hbm, ab_v)

    cid = lax.axis_index("c")
    sid = lax.axis_index("s")
    wid = sid * 2 + cid
    chunk0 = wid * CHUNKS_PER_TILE
    samp0 = chunk0 * CHUNK_SAMP

    idxs = (idx0, idx1)
    ws = (w0, w1)
    rows = (rows0, rows1)
    outs = (outb0, outb1)
    sems = (sem0, sem1)
    osems = (osem0, osem1)

    def start(p):
        pltpu.make_async_copy(patch_hbm.at[idxs[p]], rows[p], sems[p]).start()

    def wait(p):
        pltpu.make_async_copy(patch_hbm.at[idxs[p]], rows[p], sems[p]).wait()

    def out_copy(j, p):
        row0 = (chunk0 + j) * CHUNK_BINS
        return pltpu.make_async_copy(
            outs[p], out_hbm.at[pl.ds(row0, CHUNK_BINS)], osems[p])

    def drain(j, p):
        wait(p)
        @pl.when(j >= 2)
        def _():
            out_copy(j - 2, p).wait()
        _accumulate(rows[p], ws[p], outs[p])
        out_copy(j, p).start()

    _coords_weights(samp0, params_v, ab_v, idxs[0], ws[0])
    start(0)

    def pair_body(i, carry):
        for p in (0, 1):
            j = i * 2 + p
            _coords_weights(samp0 + (j + 1) * CHUNK_SAMP,
                            params_v, ab_v, idxs[1 - p], ws[1 - p])
            start(1 - p)
            drain(j, p)
        return carry

    lax.fori_loop(0, (CHUNKS_PER_TILE - 2) // 2, pair_body, 0)

    j = CHUNKS_PER_TILE - 2
    _coords_weights(samp0 + (j + 1) * CHUNK_SAMP, params_v, ab_v, idxs[1], ws[1])
    start(1)
    drain(j, 0)
    drain(j + 1, 1)
    out_copy(j, 0).wait()
    out_copy(j + 1, 1).wait()


_sc_call = functools.partial(
    pl.kernel,
    mesh=plsc.VectorSubcoreMesh(core_axis_name="c", subcore_axis_name="s"),
    compiler_params=pltpu.CompilerParams(needs_layout_passes=False),
    out_type=jax.ShapeDtypeStruct((BINS_PAD, C), _f32),
    scratch_types=[
        pltpu.VMEM((8, N_PAD), _f32),
        pltpu.VMEM((2, AB_PAD), _f32),
        pltpu.VMEM((CHUNK_SAMP,), _i32),
        pltpu.VMEM((CHUNK_SAMP,), _i32),
        pltpu.VMEM((CHUNK_BINS * 16,), _f32),
        pltpu.VMEM((CHUNK_BINS * 16,), _f32),
        pltpu.VMEM((CHUNK_SAMP, 2 * C), _i32),
        pltpu.VMEM((CHUNK_SAMP, 2 * C), _i32),
        pltpu.VMEM((CHUNK_BINS, C), _f32),
        pltpu.VMEM((CHUNK_BINS, C), _f32),
        pltpu.SemaphoreType.DMA,
        pltpu.SemaphoreType.DMA,
        pltpu.SemaphoreType.DMA,
        pltpu.SemaphoreType.DMA,
    ],
)(_sc_body)


def _ab_table():
    ab = np.zeros((2, AB_PAD), np.float32)
    for s in range(SAMP_PER_ROI):
        ph = s // (PW * GH * GW)
        pw = (s // (GH * GW)) % PW
        iy = (s // GW) % GH
        ix = s % GW
        ab[0, s] = ph + (iy + 0.5) / GH
        ab[1, s] = pw + (ix + 0.5) / GW
    return ab


_AB = _ab_table()


def kernel(input0, rois0):
    f = input0[0].transpose(1, 2, 0)
    fx = jnp.concatenate([f[:, 1:], f[:, -1:]], axis=1)
    fy = jnp.concatenate([f[1:], f[-1:]], axis=0)
    fxy = jnp.concatenate([fx[1:], fx[-1:]], axis=0)
    patch = jnp.concatenate([f, fx, fy, fxy], axis=-1).reshape(H * W, 4 * C)
    pb = patch.astype(jnp.bfloat16).reshape(H * W, 8, 2, 16)
    pb = pb.transpose(0, 1, 3, 2).reshape(H * W, 2 * C, 2)
    patch = lax.bitcast_convert_type(pb, _i32)

    ch = rois0[:, 1] * 0.5 - 0.5
    cw = rois0[:, 2] * 0.5 - 0.5
    rh = jnp.maximum(rois0[:, 4] * 0.5, 1.0)
    rw = jnp.maximum(rois0[:, 5] * 0.5, 1.0)
    theta = rois0[:, 7]
    params = jnp.stack([
        ch, cw, jnp.cos(theta), jnp.sin(theta),
        rh / PH, rw / PW, rh * 0.5, rw * 0.5,
    ])
    pad_col = jnp.array([0.0, 0.0, 1.0, 0.0, 1.0, 1.0, 3.5, 3.5], _f32)
    params = jnp.concatenate(
        [params, jnp.broadcast_to(pad_col[:, None], (8, N_PAD - NROI))], axis=1)

    out = _sc_call(patch, params, jnp.asarray(_AB))
    out = out[:BINS].reshape(NROI, PH, PW, C).transpose(0, 3, 1, 2)
    return out


# speedup vs baseline: 1.0291x; 1.0291x over previous
import functools
import math

import numpy as np
import jax
import jax.numpy as jnp
from jax import lax
from jax.experimental import pallas as pl
from jax.experimental.pallas import tpu as pltpu
from jax.experimental.pallas import tpu_sc as plsc

H, W, C = 200, 176, 64
NROI = 5000
PH = PW = 7
GH = GW = 2
SAMP_PER_ROI = PH * PW * GH * GW
BINS = NROI * PH * PW
NTILES = 32
CHUNK_BINS = 32
CHUNK_SAMP = CHUNK_BINS * GH * GW
NGROUP = CHUNK_SAMP // 16
CHUNKS_TOTAL = -(-BINS // CHUNK_BINS)
CHUNKS_PER_TILE = -(-CHUNKS_TOTAL // NTILES)
BINS_PAD = CHUNKS_PER_TILE * NTILES * CHUNK_BINS
N_PAD = (BINS_PAD * GH * GW - 1) // SAMP_PER_ROI + 1
AB_PAD = 208

_f32 = jnp.float32
_i32 = jnp.int32


def _splat_i32(v):
    return jnp.full((16,), v, dtype=_i32)


def _coords_weights(samp_base, params_v, ab_v, idx_v, w_v):
    lane = lax.iota(_i32, 16)
    for g in range(NGROUP):
        s = samp_base + g * 16 + lane
        roi = lax.div(s, _i32(SAMP_PER_ROI))
        rem = s - roi * SAMP_PER_ROI

        def p(k):
            return plsc.load_gather(params_v, [_splat_i32(k), roi])

        ch, cw, co, si, bh, bw, rh2, rw2 = (p(k) for k in range(8))
        a = plsc.load_gather(ab_v, [_splat_i32(0), rem])
        b = plsc.load_gather(ab_v, [_splat_i32(1), rem])
        yy = a * bh - rh2
        xx = b * bw - rw2
        y = yy * co - xx * si + ch
        x = yy * si + xx * co + cw
        valid = (y > -1.0) & (y < float(H)) & (x > -1.0) & (x < float(W))
        yc = jnp.maximum(y, 0.0)
        xc = jnp.maximum(x, 0.0)
        yl = jnp.minimum(yc.astype(_i32), H - 1)
        xl = jnp.minimum(xc.astype(_i32), W - 1)
        ly = yc - yl.astype(_f32)
        lx = xc - xl.astype(_f32)
        hy = 1.0 - ly
        hx = 1.0 - lx
        vf = jnp.where(valid, _f32(0.25), _f32(0.0))
        idx_v[pl.ds(g * 16, 16)] = yl * W + xl
        s_local = g * 16 + lane
        b_local = lax.div(s_local, _i32(4))
        pos = b_local * 16 + (s_local - b_local * 4) * 4
        plsc.store_scatter(w_v, [pos], hy * hx * vf)
        plsc.store_scatter(w_v, [pos + 1], hy * lx * vf)
        plsc.store_scatter(w_v, [pos + 2], ly * hx * vf)
        plsc.store_scatter(w_v, [pos + 3], ly * lx * vf)


def _accumulate(rows_v, w_v, out_v):
    @plsc.parallel_loop(0, CHUNK_BINS, unroll=2)
    def _bin_body(b):
        accs = [jnp.zeros((16,), _f32) for _ in range(4)]
        wvec = w_v[pl.ds(b * 16, 16)]
        for q in range(GH * GW):
            s = b * 4 + q
            for k in range(4):
                w = wvec[q * 4 + k]
                for h in range(2):
                    v16 = rows_v[s, pl.ds(k * 32 + h * 16, 16)]
                    vbf = plsc.bitcast(v16, jnp.bfloat16)
                    va, vb = plsc.unpack(vbf, format=plsc.PackFormat.INTERLEAVED)
                    accs[2 * h] = accs[2 * h] + w * va
                    accs[2 * h + 1] = accs[2 * h + 1] + w * vb
        for c4 in range(4):
            out_v[b, pl.ds(c4 * 16, 16)] = accs[c4]


def _sc_body(patch_hbm, params_hbm, ab_hbm, out_hbm,
             params_v, ab_v, idx0, idx1, w0, w1, rows0, rows1,
             outb0, outb1, sem0, sem1, osem0, osem1):
    pltpu.sync_copy(params_hbm, params_v)
    pltpu.sync_copy(ab_hbm, ab_v)

    cid = lax.axis_index("c")
    sid = lax.axis_index("s")
    wid = sid * 2 + cid
    chunk0 = wid * CHUNKS_PER_TILE
    samp0 = chunk0 * CHUNK_SAMP

    idxs = (idx0, idx1)
    ws = (w0, w1)
    rows = (rows0, rows1)
    outs = (outb0, outb1)
    sems = (sem0, sem1)
    osems = (osem0, osem1)

    def start(p):
        pltpu.make_async_copy(patch_hbm.at[idxs[p]], rows[p], sems[p]).start()

    def wait(p):
        pltpu.make_async_copy(patch_hbm.at[idxs[p]], rows[p], sems[p]).wait()

    def out_copy(j, p):
        row0 = (chunk0 + j) * CHUNK_BINS
        return pltpu.make_async_copy(
            outs[p], out_hbm.at[pl.ds(row0, CHUNK_BINS)], osems[p])

    def drain(j, p):
        wait(p)
        @pl.when(j >= 2)
        def _():
            out_copy(j - 2, p).wait()
        _accumulate(rows[p], ws[p], outs[p])
        out_copy(j, p).start()

    _coords_weights(samp0, params_v, ab_v, idxs[0], ws[0])
    start(0)

    def pair_body(i, carry):
        for p in (0, 1):
            j = i * 2 + p
            _coords_weights(samp0 + (j + 1) * CHUNK_SAMP,
                            params_v, ab_v, idxs[1 - p], ws[1 - p])
            start(1 - p)
            drain(j, p)
        return carry

    lax.fori_loop(0, (CHUNKS_PER_TILE - 2) // 2, pair_body, 0)

    j = CHUNKS_PER_TILE - 2
    _coords_weights(samp0 + (j + 1) * CHUNK_SAMP, params_v, ab_v, idxs[1], ws[1])
    start(1)
    drain(j, 0)
    drain(j + 1, 1)
    out_copy(j, 0).wait()
    out_copy(j + 1, 1).wait()


_sc_call = functools.partial(
    pl.kernel,
    mesh=plsc.VectorSubcoreMesh(core_axis_name="c", subcore_axis_name="s"),
    compiler_params=pltpu.CompilerParams(needs_layout_passes=False),
    out_type=jax.ShapeDtypeStruct((BINS_PAD, C), _f32),
    scratch_types=[
        pltpu.VMEM((8, N_PAD), _f32),
        pltpu.VMEM((2, AB_PAD), _f32),
        pltpu.VMEM((CHUNK_SAMP,), _i32),
        pltpu.VMEM((CHUNK_SAMP,), _i32),
        pltpu.VMEM((CHUNK_BINS * 16,), _f32),
        pltpu.VMEM((CHUNK_BINS * 16,), _f32),
        pltpu.VMEM((CHUNK_SAMP, 2 * C), _i32),
        pltpu.VMEM((CHUNK_SAMP, 2 * C), _i32),
        pltpu.VMEM((CHUNK_BINS, C), _f32),
        pltpu.VMEM((CHUNK_BINS, C), _f32),
        pltpu.SemaphoreType.DMA,
        pltpu.SemaphoreType.DMA,
        pltpu.SemaphoreType.DMA,
        pltpu.SemaphoreType.DMA,
    ],
)(_sc_body)


def _ab_table():
    ab = np.zeros((2, AB_PAD), np.float32)
    for s in range(SAMP_PER_ROI):
        ph = s // (PW * GH * GW)
        pw = (s // (GH * GW)) % PW
        iy = (s // GW) % GH
        ix = s % GW
        ab[0, s] = ph + (iy + 0.5) / GH
        ab[1, s] = pw + (ix + 0.5) / GW
    return ab


_AB = _ab_table()


def kernel(input0, rois0):
    f = input0[0].transpose(1, 2, 0)
    fx = jnp.concatenate([f[:, 1:], f[:, -1:]], axis=1)
    fy = jnp.concatenate([f[1:], f[-1:]], axis=0)
    fxy = jnp.concatenate([fx[1:], fx[-1:]], axis=0)
    patch = jnp.concatenate([f, fx, fy, fxy], axis=-1).reshape(H * W, 4 * C)
    pb = patch.astype(jnp.bfloat16).reshape(H * W, 8, 2, 16)
    pb = pb.transpose(0, 1, 3, 2).reshape(H * W, 2 * C, 2)
    patch = lax.bitcast_convert_type(pb, _i32)

    ch = rois0[:, 1] * 0.5 - 0.5
    cw = rois0[:, 2] * 0.5 - 0.5
    rh = jnp.maximum(rois0[:, 4] * 0.5, 1.0)
    rw = jnp.maximum(rois0[:, 5] * 0.5, 1.0)
    theta = rois0[:, 7]
    params = jnp.stack([
        ch, cw, jnp.cos(theta), jnp.sin(theta),
        rh / PH, rw / PW, rh * 0.5, rw * 0.5,
    ])
    pad_col = jnp.array([0.0, 0.0, 1.0, 0.0, 1.0, 1.0, 3.5, 3.5], _f32)
    params = jnp.concatenate(
        [params, jnp.broadcast_to(pad_col[:, None], (8, N_PAD - NROI))], axis=1)

    out = _sc_call(patch, params, jnp.asarray(_AB))
    out = out[:BINS].reshape(NROI, PH, PW, C).transpose(0, 3, 1, 2)
    return out
